# initial kernel scaffold (unmeasured)
import jax
import jax.numpy as jnp
from jax import lax
from jax.experimental import pallas as pl
from jax.experimental.pallas import tpu as pltpu

N_DEV = 4
EPS = 1e-5


def kernel(x, gamma, beta):
    m, n = x.shape
    n_global = n * N_DEV

    def body(x_ref, g_ref, b_ref, out_ref, loc_ref, rbuf_ref, send_sems, recv_sems):
        my = lax.axis_index("i")

        xv = x_ref[:, :]
        ones = jnp.ones((1, n), jnp.float32)
        dims = (((1,), (1,)), ((), ()))
        s = lax.dot_general(ones, xv, dims, preferred_element_type=jnp.float32)
        q = lax.dot_general(ones, xv * xv, dims, preferred_element_type=jnp.float32)
        loc_ref[0:1, :] = s
        loc_ref[1:2, :] = q

        sends = []
        for off in (1, 2, 3):
            peer = (my + off) % N_DEV
            rdma = pltpu.make_async_remote_copy(
                src_ref=loc_ref,
                dst_ref=rbuf_ref.at[3 - off],
                send_sem=send_sems.at[off - 1],
                recv_sem=recv_sems.at[3 - off],
                device_id=(peer,),
                device_id_type=pl.DeviceIdType.MESH,
            )
            rdma.start()
            sends.append(rdma)

        for k in range(3):
            recv = pltpu.make_async_remote_copy(
                src_ref=loc_ref,
                dst_ref=rbuf_ref.at[k],
                send_sem=send_sems.at[0],
                recv_sem=recv_sems.at[k],
                device_id=(my,),
                device_id_type=pl.DeviceIdType.MESH,
            )
            recv.wait_recv()
        for rdma in sends:
            rdma.wait_send()

        tot = loc_ref[:, :] + rbuf_ref[0] + rbuf_ref[1] + rbuf_ref[2]
        mean_r = tot[0:1, :] / n_global
        var_r = tot[1:2, :] / n_global - mean_r * mean_r
        rstd_r = lax.rsqrt(var_r + EPS)
        mv = jnp.concatenate([mean_r, rstd_r], axis=0)
        t = mv.T
        mean_c = t[:, 0:1]
        rstd_c = t[:, 1:2]
        out_ref[:, :] = (xv - mean_c) * rstd_c * g_ref[:, :] + b_ref[:, :]

    return pl.pallas_call(
        body,
        out_shape=jax.ShapeDtypeStruct((m, n), jnp.float32),
        in_specs=[pl.BlockSpec(memory_space=pltpu.VMEM)] * 3,
        out_specs=pl.BlockSpec(memory_space=pltpu.VMEM),
        scratch_shapes=[
            pltpu.VMEM((2, m), jnp.float32),
            pltpu.VMEM((3, 2, m), jnp.float32),
            pltpu.SemaphoreType.DMA((3,)),
            pltpu.SemaphoreType.DMA((3,)),
        ],
        compiler_params=pltpu.CompilerParams(collective_id=0),
    )(x, gamma.reshape(1, n), beta.reshape(1, n))


# baseline (device time: 20885 ns/iter reference)
import jax
import jax.numpy as jnp
from jax import lax
from jax.experimental import pallas as pl
from jax.experimental.pallas import tpu as pltpu

N_DEV = 4
EPS = 1e-5


def kernel(x, gamma, beta):
    m, n = x.shape
    n_global = n * N_DEV

    def body(x_ref, g_ref, b_ref, out_ref, loc_ref, rbuf_ref, send_sems, recv_sems):
        my = lax.axis_index("i")

        xv = x_ref[:, :]
        ones = jnp.ones((1, n), jnp.float32)
        dims = (((1,), (1,)), ((), ()))
        s = lax.dot_general(ones, xv, dims, preferred_element_type=jnp.float32)
        q = lax.dot_general(ones, xv * xv, dims, preferred_element_type=jnp.float32)
        loc_ref[0:1, :] = s
        loc_ref[1:2, :] = q

        sends = []
        for off in (1, 2, 3):
            peer = (my + off) % N_DEV
            rdma = pltpu.make_async_remote_copy(
                src_ref=loc_ref,
                dst_ref=rbuf_ref.at[3 - off],
                send_sem=send_sems.at[off - 1],
                recv_sem=recv_sems.at[3 - off],
                device_id=(peer,),
                device_id_type=pl.DeviceIdType.MESH,
            )
            rdma.start()
            sends.append(rdma)

        for k in range(3):
            recv = pltpu.make_async_remote_copy(
                src_ref=loc_ref,
                dst_ref=rbuf_ref.at[k],
                send_sem=send_sems.at[0],
                recv_sem=recv_sems.at[k],
                device_id=(my,),
                device_id_type=pl.DeviceIdType.MESH,
            )
            recv.wait_recv()
        for rdma in sends:
            rdma.wait_send()

        tot = loc_ref[:, :] + rbuf_ref[0] + rbuf_ref[1] + rbuf_ref[2]
        mean_r = tot[0:1, :] / n_global
        var_r = tot[1:2, :] / n_global - mean_r * mean_r
        rstd_r = lax.rsqrt(var_r + EPS)
        mv = jnp.concatenate([mean_r, rstd_r], axis=0)
        t = mv.T
        mean_c = t[:, 0:1]
        rstd_c = t[:, 1:2]
        out_ref[:, :] = (xv - mean_c) * rstd_c * g_ref[:, :] + b_ref[:, :]

    return pl.pallas_call(
        body,
        out_shape=jax.ShapeDtypeStruct((m, n), jnp.float32),
        in_specs=[pl.BlockSpec(memory_space=pltpu.VMEM)] * 3,
        out_specs=pl.BlockSpec(memory_space=pltpu.VMEM),
        scratch_shapes=[
            pltpu.VMEM((2, m), jnp.float32),
            pltpu.VMEM((3, 2, m), jnp.float32),
            pltpu.SemaphoreType.DMA((3,)),
            pltpu.SemaphoreType.DMA((3,)),
        ],
    )(x, gamma.reshape(1, n), beta.reshape(1, n))
